# Initial kernel scaffold; baseline (speedup 1.0000x reference)
import jax
import jax.numpy as jnp
from jax.experimental import pallas as pl


def kernel(cat_input, num_input, tables, W1, b1, W2, b2):
    raise NotImplementedError("write your pallas kernel here")


# speedup vs baseline: 7.8812x; 7.8812x over previous
import functools

import jax
import jax.numpy as jnp
from jax import lax
from jax.experimental import pallas as pl
from jax.experimental.pallas import tpu as pltpu
from jax.experimental.pallas import tpu_sc as plsc

F = 26
V = 100000
D = 32
B = 16384
NUM_NUMERIC = 13
H1 = 256
H2 = 64

GB = 128
CH = 1024
NG = CH // GB


def _sc_gather(cat_flat_2d, tables_flat):
    info = plsc.get_sparse_core_info()
    NC, NS = info.num_cores, info.num_subcores
    NW = NC * NS
    TOT = cat_flat_2d.shape[0] * GB
    per_w = TOT // NW
    nch = per_w // CH

    @functools.partial(
        pl.kernel,
        mesh=plsc.VectorSubcoreMesh(core_axis_name="c", subcore_axis_name="s"),
        out_type=jax.ShapeDtypeStruct((TOT, D), jnp.float32),
        scratch_types=[
            pltpu.VMEM((NG, GB), jnp.int32),
            pltpu.VMEM((CH, D), jnp.float32),
            pltpu.SemaphoreType.DMA,
        ],
        compiler_params=pltpu.CompilerParams(use_tc_tiling_on_sc=False),
    )
    def gather_k(cat_hbm, tab_hbm, out_hbm, idx_v, rows_v, sem):
        wid = lax.axis_index("s") * NC + lax.axis_index("c")
        lane = lax.iota(jnp.int32, 16)

        @pl.loop(0, nch)
        def _chunk(c):
            base = pl.multiple_of(wid * per_w + c * CH, CH)
            pltpu.sync_copy(cat_hbm.at[pl.ds(pl.multiple_of(base // GB, 8), NG)], idx_v)

            @pl.loop(0, NG)
            def _row(r):
                @pl.loop(0, GB // 16)
                def _vec(i):
                    pos = base + r * GB + i * 16 + lane
                    off = (pos % F) * V
                    sl = (r, pl.ds(i * 16, 16))
                    idx_v[sl] = idx_v[sl] + off

            copies = [
                pltpu.async_copy(
                    tab_hbm.at[idx_v.at[r]],
                    rows_v.at[pl.ds(r * GB, GB)],
                    sem,
                )
                for r in range(NG)
            ]
            for cp in copies:
                cp.wait()
            pltpu.sync_copy(rows_v, out_hbm.at[pl.ds(base, CH)])

    return gather_k(cat_flat_2d, tables_flat)


def _tc_mlp(embeds, num_pad, w1a, w1b, b1, w2, b2):
    bb = 512
    grid = (B // bb,)
    kin = F * D

    def body(x_ref, n_ref, w1a_ref, w1b_ref, b1_ref, w2_ref, b2_ref, o_ref):
        h = jnp.dot(x_ref[...], w1a_ref[...], preferred_element_type=jnp.float32)
        h += jnp.dot(n_ref[...], w1b_ref[...], preferred_element_type=jnp.float32)
        h = jnp.maximum(h + b1_ref[...], 0.0)
        o = jnp.dot(h, w2_ref[...], preferred_element_type=jnp.float32) + b2_ref[...]
        o_ref[...] = jnp.maximum(o, 0.0)

    return pl.pallas_call(
        body,
        grid=grid,
        in_specs=[
            pl.BlockSpec((bb, kin), lambda i: (i, 0)),
            pl.BlockSpec((bb, 16), lambda i: (i, 0)),
            pl.BlockSpec((kin, H1), lambda i: (0, 0)),
            pl.BlockSpec((16, H1), lambda i: (0, 0)),
            pl.BlockSpec((1, H1), lambda i: (0, 0)),
            pl.BlockSpec((H1, H2), lambda i: (0, 0)),
            pl.BlockSpec((1, H2), lambda i: (0, 0)),
        ],
        out_specs=pl.BlockSpec((bb, H2), lambda i: (i, 0)),
        out_shape=jax.ShapeDtypeStruct((B, H2), jnp.float32),
        compiler_params=pltpu.CompilerParams(
            dimension_semantics=("arbitrary",),
        ),
    )(embeds, num_pad, w1a, w1b, b1, w2, b2)


def kernel(cat_input, num_input, tables, W1, b1, W2, b2):
    cat_flat = cat_input.reshape(-1)
    cat_2d = cat_flat.reshape(-1, GB)
    tables_flat = tables.reshape(F * V, D)

    embeds = _sc_gather(cat_2d, tables_flat)
    embeds = embeds.reshape(B, F * D)

    num_pad = jnp.pad(num_input, ((0, 0), (0, 16 - NUM_NUMERIC)))
    w1a = W1[: F * D]
    w1b = jnp.pad(W1[F * D :], ((0, 16 - NUM_NUMERIC), (0, 0)))
    return _tc_mlp(embeds, num_pad, w1a, w1b,
                   b1.reshape(1, H1), W2, b2.reshape(1, H2))
